# Initial kernel scaffold; baseline (speedup 1.0000x reference)
import jax
import jax.numpy as jnp
from jax.experimental import pallas as pl


def kernel(energy, species_idx, shifts, scales):
    raise NotImplementedError("write your pallas kernel here")


# speedup vs baseline: 33.7797x; 33.7797x over previous
import jax
import jax.numpy as jnp
from jax import lax
from jax.experimental import pallas as pl
from jax.experimental.pallas import tpu as pltpu
from jax.experimental.pallas import tpu_sc as plsc

_T_PAD = 128
_C = 800
_LANES = 16


def _make_body(n_chunks, n_workers, num_cores):
    max_chunks_per_w = -(-n_chunks // n_workers)

    def _body(e_hbm, idx_hbm, sh_hbm, sc_hbm, out_hbm, sh_v, sc_v, idx_v, e_v, o_v):
        wid = lax.axis_index("s") * num_cores + lax.axis_index("c")
        pltpu.sync_copy(sh_hbm, sh_v)
        pltpu.sync_copy(sc_hbm, sc_v)
        for t in range(max_chunks_per_w):
            chunk = wid + t * n_workers

            @pl.when(chunk < n_chunks)
            def _():
                base = chunk * _C
                pltpu.sync_copy(e_hbm.at[pl.ds(base, _C)], e_v)
                pltpu.sync_copy(idx_hbm.at[pl.ds(base, _C)], idx_v)
                for j in range(_C // _LANES):
                    sl = pl.ds(j * _LANES, _LANES)
                    iv = idx_v[sl]
                    sv = plsc.load_gather(sh_v, [iv])
                    cv = plsc.load_gather(sc_v, [iv])
                    o_v[sl] = e_v[sl] * cv + sv
                pltpu.sync_copy(o_v, out_hbm.at[pl.ds(base, _C)])

    return _body


def kernel(energy, species_idx, shifts, scales):
    n = energy.shape[0]
    assert n % _C == 0
    n_chunks = n // _C
    e = energy.reshape(n)
    sh = jnp.zeros((_T_PAD,), jnp.float32).at[: shifts.shape[0]].set(shifts)
    sc = jnp.zeros((_T_PAD,), jnp.float32).at[: scales.shape[0]].set(scales)
    mesh = plsc.VectorSubcoreMesh(
        core_axis_name="c", subcore_axis_name="s", num_cores=2, num_subcores=16
    )
    n_workers = mesh.num_cores * mesh.num_subcores
    run = pl.kernel(
        _make_body(n_chunks, n_workers, mesh.num_cores),
        out_type=jax.ShapeDtypeStruct((n,), jnp.float32),
        mesh=mesh,
        compiler_params=pltpu.CompilerParams(needs_layout_passes=False),
        scratch_types=[
            pltpu.VMEM((_T_PAD,), jnp.float32),
            pltpu.VMEM((_T_PAD,), jnp.float32),
            pltpu.VMEM((_C,), jnp.int32),
            pltpu.VMEM((_C,), jnp.float32),
            pltpu.VMEM((_C,), jnp.float32),
        ],
    )
    return run(e, species_idx, sh, sc).reshape(n, 1)


# speedup vs baseline: 39.7153x; 1.1757x over previous
import jax
import jax.numpy as jnp
from jax import lax
from jax.experimental import pallas as pl
from jax.experimental.pallas import tpu as pltpu
from jax.experimental.pallas import tpu_sc as plsc

_T_PAD = 128
_LANES = 16
_NW = 32


def _make_body(n, span, num_cores):
    def _body(e_hbm, idx_hbm, sh_hbm, sc_hbm, out_hbm,
              sh_v, sc_v, idx_v, e_v, o_v, sem):
        wid = lax.axis_index("s") * num_cores + lax.axis_index("c")
        base = jnp.minimum(wid * span, n - span)
        c1 = pltpu.async_copy(sh_hbm, sh_v, sem)
        c2 = pltpu.async_copy(sc_hbm, sc_v, sem)
        c3 = pltpu.async_copy(e_hbm.at[pl.ds(base, span)], e_v, sem)
        c4 = pltpu.async_copy(idx_hbm.at[pl.ds(base, span)], idx_v, sem)
        c1.wait(); c2.wait(); c3.wait(); c4.wait()
        for j in range(span // _LANES):
            sl = pl.ds(j * _LANES, _LANES)
            iv = idx_v[sl]
            sv = plsc.load_gather(sh_v, [iv])
            cv = plsc.load_gather(sc_v, [iv])
            o_v[sl] = e_v[sl] * cv + sv
        pltpu.sync_copy(o_v, out_hbm.at[pl.ds(base, span)])

    return _body


def kernel(energy, species_idx, shifts, scales):
    n = energy.shape[0]
    span = -(-n // (_NW * _LANES)) * _LANES
    assert span % 8 == 0 and (n - span) % 8 == 0 and span <= n
    e = energy.reshape(n)
    sh = jnp.zeros((_T_PAD,), jnp.float32).at[: shifts.shape[0]].set(shifts)
    sc = jnp.zeros((_T_PAD,), jnp.float32).at[: scales.shape[0]].set(scales)
    mesh = plsc.VectorSubcoreMesh(
        core_axis_name="c", subcore_axis_name="s", num_cores=2, num_subcores=16
    )
    run = pl.kernel(
        _make_body(n, span, mesh.num_cores),
        out_type=jax.ShapeDtypeStruct((n,), jnp.float32),
        mesh=mesh,
        compiler_params=pltpu.CompilerParams(needs_layout_passes=False),
        scratch_types=[
            pltpu.VMEM((_T_PAD,), jnp.float32),
            pltpu.VMEM((_T_PAD,), jnp.float32),
            pltpu.VMEM((span,), jnp.int32),
            pltpu.VMEM((span,), jnp.float32),
            pltpu.VMEM((span,), jnp.float32),
            pltpu.SemaphoreType.DMA,
        ],
    )
    return run(e, species_idx, sh, sc).reshape(n, 1)


# speedup vs baseline: 43.0221x; 1.0833x over previous
import jax
import jax.numpy as jnp
from jax import lax
from jax.experimental import pallas as pl
from jax.experimental.pallas import tpu as pltpu
from jax.experimental.pallas import tpu_sc as plsc

_T_PAD = 128
_LANES = 16
_NW = 32


def _make_body(n, span, num_cores):
    def _body(e_hbm, idx_hbm, sh_hbm, sc_hbm, out_hbm,
              sh_v, sc_v, idx_v, e_v, o_v, sem):
        wid = lax.axis_index("s") * num_cores + lax.axis_index("c")
        base = jnp.minimum(wid * span, n - span)
        c1 = pltpu.async_copy(sh_hbm, sh_v, sem)
        c2 = pltpu.async_copy(sc_hbm, sc_v, sem)
        c3 = pltpu.async_copy(e_hbm.at[pl.ds(base, span)], e_v, sem)
        c4 = pltpu.async_copy(idx_hbm.at[pl.ds(base, span)], idx_v, sem)
        c1.wait(); c2.wait(); c3.wait(); c4.wait()

        @pl.loop(0, span // _LANES, unroll=14)
        def _(j):
            sl = pl.ds(j * _LANES, _LANES)
            iv = idx_v[sl]
            sv = plsc.load_gather(sh_v, [iv])
            cv = plsc.load_gather(sc_v, [iv])
            o_v[sl] = e_v[sl] * cv + sv

        pltpu.sync_copy(o_v, out_hbm.at[pl.ds(base, span)])

    return _body


def kernel(energy, species_idx, shifts, scales):
    n = energy.shape[0]
    span = -(-n // (_NW * _LANES)) * _LANES
    assert span % 8 == 0 and (n - span) % 8 == 0 and span <= n
    e = energy.reshape(n)
    n_types = shifts.shape[0]
    mesh = plsc.VectorSubcoreMesh(
        core_axis_name="c", subcore_axis_name="s", num_cores=2, num_subcores=16
    )
    run = pl.kernel(
        _make_body(n, span, mesh.num_cores),
        out_type=jax.ShapeDtypeStruct((n,), jnp.float32),
        mesh=mesh,
        compiler_params=pltpu.CompilerParams(needs_layout_passes=False),
        scratch_types=[
            pltpu.VMEM((n_types,), jnp.float32),
            pltpu.VMEM((n_types,), jnp.float32),
            pltpu.VMEM((span,), jnp.int32),
            pltpu.VMEM((span,), jnp.float32),
            pltpu.VMEM((span,), jnp.float32),
            pltpu.SemaphoreType.DMA,
        ],
    )
    return run(e, species_idx, shifts, scales).reshape(n, 1)
